# Initial kernel scaffold; baseline (speedup 1.0000x reference)
import jax
import jax.numpy as jnp
from jax.experimental import pallas as pl


def kernel(criterionOutput, networkOutput, batch):
    raise NotImplementedError("write your pallas kernel here")


# speedup vs baseline: 300.0161x; 300.0161x over previous
import jax
import jax.numpy as jnp
from jax.experimental import pallas as pl


_BQ = 200


def _cdist_rank_kernel(q_ref, emb_ref, dists_ref, ranks_ref):
    i = pl.program_id(0)
    q = q_ref[...]
    e = emb_ref[...]
    n_cols = e.shape[0]

    qn = jnp.sum(q * q, axis=1, keepdims=True)
    en = jnp.sum(e * e, axis=1)[None, :]
    dot = jax.lax.dot_general(
        q, e, (((1,), (1,)), ((), ())),
        preferred_element_type=jnp.float32)
    sq = jnp.maximum(qn + en - 2.0 * dot, 0.0)
    d = jnp.sqrt(jnp.maximum(sq, 1e-12))
    dists_ref[...] = d

    col = jax.lax.broadcasted_iota(jnp.int32, d.shape, 1)
    row = jax.lax.broadcasted_iota(jnp.int32, d.shape, 0)
    valid = col < n_cols
    pos = 3 * (i * _BQ + row) + 1
    at_pos = (col == pos) & valid
    d_pos = jnp.sum(jnp.where(at_pos, d, 0.0), axis=1, keepdims=True)
    lt = jnp.sum(((d < d_pos) & valid).astype(jnp.int32), axis=1, keepdims=True)
    eq_before = jnp.sum(((d == d_pos) & (col < pos) & valid).astype(jnp.int32),
                        axis=1, keepdims=True)
    ranks_ref[...] = lt + eq_before - 1


def kernel(criterionOutput, networkOutput, batch):
    emb = networkOutput
    n, dim = emb.shape
    q_count = n // 3
    queries = emb[0::3][:q_count]

    grid = (q_count // _BQ,)
    dists, ranks = pl.pallas_call(
        _cdist_rank_kernel,
        grid=grid,
        in_specs=[
            pl.BlockSpec((_BQ, dim), lambda i: (i, 0)),
            pl.BlockSpec((n, dim), lambda i: (0, 0)),
        ],
        out_specs=[
            pl.BlockSpec((_BQ, n), lambda i: (i, 0)),
            pl.BlockSpec((_BQ, 1), lambda i: (i, 0)),
        ],
        out_shape=[
            jax.ShapeDtypeStruct((q_count, n), jnp.float32),
            jax.ShapeDtypeStruct((q_count, 1), jnp.int32),
        ],
    )(queries, emb)

    positive_ranks = ranks.reshape(q_count)
    medr = jnp.mean(positive_ranks.astype(jnp.float32))
    return dists, positive_ranks, medr


# speedup vs baseline: 420.8952x; 1.4029x over previous
import jax
import jax.numpy as jnp
from jax.experimental import pallas as pl
from jax.experimental.pallas import tpu as pltpu


_BQ = 200


def _cdist_rank_kernel(q_ref, p_ref, emb_ref, dists_ref, ranks_ref, en_ref):
    i = pl.program_id(0)
    e = emb_ref[...]

    @pl.when(i == 0)
    def _():
        en_ref[...] = jnp.sum(e * e, axis=1)[None, :]

    q = q_ref[...]
    p = p_ref[...]
    qn = jnp.sum(q * q, axis=1, keepdims=True)
    pn = jnp.sum(p * p, axis=1, keepdims=True)
    qp = jnp.sum(q * p, axis=1, keepdims=True)
    c_pos = jnp.maximum(qn + pn - 2.0 * qp, 1e-12)

    dot = jax.lax.dot_general(
        q, e, (((1,), (1,)), ((), ())),
        preferred_element_type=jnp.float32)
    en = en_ref[...]
    c = jnp.maximum(qn + (en - 2.0 * dot), 1e-12)
    dists_ref[...] = jnp.sqrt(c)
    lt = jnp.sum((c < c_pos).astype(jnp.int32), axis=1, keepdims=True)
    ranks_ref[...] = lt - 1


def kernel(criterionOutput, networkOutput, batch):
    emb = networkOutput
    n, dim = emb.shape
    q_count = n // 3
    queries = emb[0::3][:q_count]
    positives = emb[1::3][:q_count]

    grid = (q_count // _BQ,)
    dists, ranks = pl.pallas_call(
        _cdist_rank_kernel,
        grid=grid,
        in_specs=[
            pl.BlockSpec((_BQ, dim), lambda i: (i, 0)),
            pl.BlockSpec((_BQ, dim), lambda i: (i, 0)),
            pl.BlockSpec((n, dim), lambda i: (0, 0)),
        ],
        out_specs=[
            pl.BlockSpec((_BQ, n), lambda i: (i, 0)),
            pl.BlockSpec((_BQ, 1), lambda i: (i, 0)),
        ],
        out_shape=[
            jax.ShapeDtypeStruct((q_count, n), jnp.float32),
            jax.ShapeDtypeStruct((q_count, 1), jnp.int32),
        ],
        scratch_shapes=[pltpu.VMEM((1, n), jnp.float32)],
    )(queries, positives, emb)

    positive_ranks = ranks.reshape(q_count)
    medr = jnp.mean(positive_ranks.astype(jnp.float32))
    return dists, positive_ranks, medr
